# baseline (speedup 1.0000x reference)
import functools

import jax
import jax.numpy as jnp
from jax import lax
from jax.experimental import pallas as pl
from jax.experimental.pallas import tpu as pltpu
from jax.experimental.pallas import tpu_sc as plsc

N = 16384
D_IN = 16384
D_OUT = 64
NNZ = 2621440

NW = 32
ROWS_W = N // NW
CH = 512
CHP = CH + 16
SUB = 128
NKV = D_OUT // 16
PTRP = ROWS_W + 16

_mesh = plsc.VectorSubcoreMesh(core_axis_name="c", subcore_axis_name="s")


@functools.partial(
    pl.kernel,
    out_type=jax.ShapeDtypeStruct((N, D_OUT), jnp.float32),
    mesh=_mesh,
    compiler_params=pltpu.CompilerParams(use_tc_tiling_on_sc=False),
    scratch_types=[
        pltpu.VMEM((PTRP,), jnp.int32),
        pltpu.VMEM((2, CH), jnp.int32),
        pltpu.VMEM((2, CHP), jnp.float32),
        pltpu.VMEM((2, CH, D_OUT), jnp.float32),
        pltpu.VMEM((ROWS_W, D_OUT), jnp.float32),
        pltpu.VMEM((D_OUT,), jnp.float32),
        pltpu.SemaphoreType.DMA,
        pltpu.SemaphoreType.DMA,
        pltpu.SemaphoreType.DMA,
        pltpu.SemaphoreType.DMA,
        pltpu.SemaphoreType.DMA,
        pltpu.SemaphoreType.DMA,
    ],
)
def _spmm_sc(values_hbm, col_hbm, w_hbm, b_hbm, ptr_hbm,
             out_hbm, ptr_v, cidx_v, vals_v, rows_v, acc_v,
             b_v, sem_g0, sem_g1, sem_s0, sem_s1, sem_c0, sem_c1):
    wid = lax.axis_index("s") * 2 + lax.axis_index("c")
    row0 = wid * ROWS_W

    pltpu.sync_copy(ptr_hbm.at[pl.ds(row0, PTRP)], ptr_v)
    pltpu.sync_copy(b_hbm, b_v)

    lo = ptr_v[pl.ds(0, 16)][0]
    hi = ptr_v[pl.ds(ROWS_W - 8, 16)][8]
    lo8 = (lo // 8) * 8
    n_ch = (hi - lo8 + CH - 1) // CH
    n_pair = (n_ch + 1) // 2

    sem_g = (sem_g0, sem_g1)
    sem_s = (sem_s0, sem_s1)
    sem_c = (sem_c0, sem_c1)

    binit = [b_v[pl.ds(16 * k, 16)] for k in range(NKV)]

    @plsc.parallel_loop(0, ROWS_W)
    def _(r):
        for k in range(NKV):
            acc_v[r, pl.ds(16 * k, 16)] = binit[k]

    def chunk_start(i):
        return jnp.minimum(lo8 + i * CH, NNZ - CH)

    def issue_cidx(i, b):
        pltpu.async_copy(col_hbm.at[pl.ds(chunk_start(i), CH)],
                         cidx_v.at[b], sem_c[b])

    def wait_cidx(b):
        pltpu.make_async_copy(col_hbm.at[pl.ds(0, CH)], cidx_v.at[b],
                              sem_c[b]).wait()

    def issue_rest(i, b):
        s = chunk_start(i)
        for q in range(CH // SUB):
            pltpu.async_copy(
                w_hbm.at[cidx_v.at[b, pl.ds(q * SUB, SUB)]],
                rows_v.at[b, pl.ds(q * SUB, SUB)], sem_g[b])
        pltpu.async_copy(values_hbm.at[pl.ds(s, CH)],
                         vals_v.at[b, pl.ds(0, CH)], sem_s[b])

    def drain(b):
        for q in range(CH // SUB):
            pltpu.make_async_copy(
                w_hbm.at[pl.ds(0, SUB)],
                rows_v.at[b, pl.ds(q * SUB, SUB)], sem_g[b]).wait()
        pltpu.make_async_copy(values_hbm.at[pl.ds(0, CH)],
                              vals_v.at[b, pl.ds(0, CH)], sem_s[b]).wait()

    lane = lax.iota(jnp.int32, 16)

    def compute(i, b, cr0):
        start = lo8 + i * CH
        s = chunk_start(i)
        lo_g = jnp.maximum(lo, start)
        hi_g = jnp.minimum(hi, start + CH)

        def ptr_at(r):
            pv = ptr_v[pl.ds(r - row0, 16)]
            return pv[0]

        def bs_body(_, st):
            lo_s, hi_s = st
            mid = (lo_s + hi_s) // 2
            above = ptr_at(row0 + mid + 1) > hi_g
            return (jnp.where(above, lo_s, mid + 1),
                    jnp.where(above, mid, hi_s))

        la, _ = lax.fori_loop(0, 10, bs_body, (0, ROWS_W))
        cr_end = row0 + la
        partial = (cr_end < row0 + ROWS_W) & (ptr_at(cr_end) < hi_g)
        n_proc = cr_end - cr0 + jnp.where(partial, 1, 0)

        def row_body(q, _):
            r = cr0 + q
            pv = ptr_v[pl.ds(r - row0, 16)]
            jlo = jnp.maximum(pv[0], lo_g) - s
            jhi = jnp.minimum(pv[1], hi_g) - s
            rc = r - row0
            g0 = (jlo // 16) * 16

            @plsc.parallel_loop(g0, jhi, step=16)
            def _(j0):
                idxv = j0 + lane
                m = (idxv >= jlo) & (idxv < jhi)
                vvm = jnp.where(m, vals_v[b, pl.ds(j0, 16)], 0.0)
                for k in range(NKV):
                    ps = [vvm[t] * rows_v[b, j0 + t, pl.ds(16 * k, 16)]
                          for t in range(16)]
                    while len(ps) > 1:
                        ps = [ps[i2] + ps[i2 + 1]
                              for i2 in range(0, len(ps), 2)]
                    plsc.addupdate(acc_v.at[rc, pl.ds(16 * k, 16)],
                                   ps[0])
            return 0

        lax.fori_loop(0, n_proc, row_body, 0)
        return cr_end

    def step(i, b, cr):
        nb = 1 - b
        wait_cidx(nb)
        issue_rest(i + 1, nb)
        drain(b)
        issue_cidx(i + 2, b)
        return compute(i, b, cr)

    pltpu.sync_copy(col_hbm.at[pl.ds(chunk_start(0), CH)], cidx_v.at[0])
    issue_rest(0, 0)
    issue_cidx(1, 1)

    def pair_body(p, cr):
        cr = step(2 * p, 0, cr)
        cr = step(2 * p + 1, 1, cr)
        return cr

    lax.fori_loop(0, n_pair, pair_body, row0)

    wait_cidx(1)
    drain(0)

    pltpu.sync_copy(acc_v, out_hbm.at[pl.ds(row0, ROWS_W)])


def kernel(values, row_idx, col_idx, W, b):
    edges = jnp.arange(0, N + 1, dtype=jnp.int32)
    row_ptr = jnp.searchsorted(row_idx, edges).astype(jnp.int32)
    pad = (N - ROWS_W + PTRP) - (N + 1)
    row_ptr = jnp.concatenate([row_ptr, jnp.full((pad,), NNZ, jnp.int32)])
    return _spmm_sc(values, col_idx, W, b, row_ptr)


# speedup vs baseline: 17.4784x; 17.4784x over previous
import functools

import jax
import jax.numpy as jnp
from jax import lax
from jax.experimental import pallas as pl
from jax.experimental.pallas import tpu as pltpu
from jax.experimental.pallas import tpu_sc as plsc

N = 16384
D_IN = 16384
D_OUT = 64
NNZ = 2621440

NW = 32
ROWS_W = N // NW
CH = 512
CHP = CH + 16
SUB = 128
NKV = D_OUT // 16
BND = CH + 48

_mesh = plsc.VectorSubcoreMesh(core_axis_name="c", subcore_axis_name="s")


@functools.partial(
    pl.kernel,
    out_type=jax.ShapeDtypeStruct((N, D_OUT), jnp.float32),
    mesh=_mesh,
    compiler_params=pltpu.CompilerParams(use_tc_tiling_on_sc=False,
                                         needs_layout_passes=False),
    scratch_types=[
        pltpu.VMEM((48,), jnp.int32),
        pltpu.VMEM((2, CH), jnp.int32),
        pltpu.VMEM((2, CHP), jnp.int32),
        pltpu.VMEM((2, CHP), jnp.float32),
        pltpu.VMEM((2, CH, D_OUT), jnp.float32),
        pltpu.VMEM((ROWS_W, D_OUT), jnp.float32),
        pltpu.VMEM((D_OUT,), jnp.float32),
        pltpu.VMEM((BND,), jnp.int32),
        pltpu.SemaphoreType.DMA,
        pltpu.SemaphoreType.DMA,
        pltpu.SemaphoreType.DMA,
        pltpu.SemaphoreType.DMA,
        pltpu.SemaphoreType.DMA,
        pltpu.SemaphoreType.DMA,
    ],
)
def _spmm_sc(values_hbm, row_hbm, col_hbm, w_hbm, b_hbm, bounds_hbm,
             out_hbm, bounds_v, cidx_v, ridx_v, vals_v, rows_v, acc_v,
             b_v, bnd_v, sem_g0, sem_g1, sem_s0, sem_s1, sem_c0, sem_c1):
    wid = lax.axis_index("s") * 2 + lax.axis_index("c")
    row0 = wid * ROWS_W

    pltpu.sync_copy(bounds_hbm, bounds_v)
    pltpu.sync_copy(b_hbm, b_v)

    bv = bounds_v[pl.ds(wid, 16)]
    lo = bv[0]
    hi = bv[1]
    lo8 = (lo // 8) * 8
    n_ch = (hi - lo8 + CH - 1) // CH
    n_pair = (n_ch + 1) // 2

    sem_g = (sem_g0, sem_g1)
    sem_s = (sem_s0, sem_s1)
    sem_c = (sem_c0, sem_c1)

    binit = [b_v[pl.ds(16 * k, 16)] for k in range(NKV)]

    @plsc.parallel_loop(0, ROWS_W)
    def _(r):
        for k in range(NKV):
            acc_v[r, pl.ds(16 * k, 16)] = binit[k]

    def chunk_start(i):
        return jnp.minimum(lo8 + i * CH, NNZ - CH)

    def issue_cidx(i, b):
        pltpu.async_copy(col_hbm.at[pl.ds(chunk_start(i), CH)],
                         cidx_v.at[b], sem_c[b])

    def wait_cidx(b):
        pltpu.make_async_copy(col_hbm.at[pl.ds(0, CH)], cidx_v.at[b],
                              sem_c[b]).wait()

    def issue_rest(i, b):
        s = chunk_start(i)
        for q in range(CH // SUB):
            pltpu.async_copy(
                w_hbm.at[cidx_v.at[b, pl.ds(q * SUB, SUB)]],
                rows_v.at[b, pl.ds(q * SUB, SUB)], sem_g[b])
        pltpu.async_copy(values_hbm.at[pl.ds(s, CH)],
                         vals_v.at[b, pl.ds(0, CH)], sem_s[b])
        pltpu.async_copy(row_hbm.at[pl.ds(s, CH)],
                         ridx_v.at[b, pl.ds(0, CH)], sem_s[b])

    def drain(b):
        for q in range(CH // SUB):
            pltpu.make_async_copy(
                w_hbm.at[pl.ds(0, SUB)],
                rows_v.at[b, pl.ds(q * SUB, SUB)], sem_g[b]).wait()
        pltpu.make_async_copy(values_hbm.at[pl.ds(0, CH)],
                              vals_v.at[b, pl.ds(0, CH)], sem_s[b]).wait()
        pltpu.make_async_copy(row_hbm.at[pl.ds(0, CH)],
                              ridx_v.at[b, pl.ds(0, CH)], sem_s[b]).wait()

    lane = lax.iota(jnp.int32, 16)

    def compute(i, b):
        start = lo8 + i * CH
        s = chunk_start(i)
        w_lo = jnp.clip(jnp.maximum(lo, start) - s, 0, CH)
        w_hi = jnp.clip(jnp.minimum(hi, start + CH) - s, 0, CH)

        bnd_v[pl.ds(0, 16)] = jnp.full((16,), w_lo, jnp.int32)

        def bdy(g, cnt):
            j0 = g * 16
            rv = ridx_v[b, pl.ds(j0, 16)]
            rnx = ridx_v[b, pl.ds(j0 + 1, 16)]
            pos = j0 + 1 + lane
            m = (rnx != rv) & (pos > w_lo) & (pos < w_hi)
            key = jnp.where(m, pos, jnp.int32(2 * CH))
            ks, _ = plsc.sort_key_val(key, key)
            bnd_v[pl.ds(cnt + 1, 16)] = ks
            c = plsc.all_reduce_population_count(m)
            return cnt + c[0]

        cnt = lax.fori_loop(0, CH // 16, bdy, 0)
        bnd_v[pl.ds(cnt + 1, 16)] = jnp.full((16,), w_hi, jnp.int32)

        def run_body(q, _):
            bb = bnd_v[pl.ds(q, 16)]
            p0 = bb[0]
            p1 = bb[1]
            rr = ridx_v[b, pl.ds(p0, 16)][0]
            rc = jnp.clip(rr - row0, 0, ROWS_W - 1)
            g0 = (p0 // 16) * 16

            @plsc.parallel_loop(g0, p1, step=16)
            def _(j0):
                idxv = j0 + lane
                m2 = (idxv >= p0) & (idxv < p1)
                vvm = jnp.where(m2, vals_v[b, pl.ds(j0, 16)], 0.0)
                for k in range(NKV):
                    ps = [vvm[t] * rows_v[b, j0 + t, pl.ds(16 * k, 16)]
                          for t in range(16)]
                    while len(ps) > 1:
                        ps = [ps[i2] + ps[i2 + 1]
                              for i2 in range(0, len(ps), 2)]
                    plsc.addupdate(acc_v.at[rc, pl.ds(16 * k, 16)],
                                   ps[0])
            return 0

        lax.fori_loop(0, cnt + 1, run_body, 0)

    def step(i, b):
        nb = 1 - b
        wait_cidx(nb)
        issue_rest(i + 1, nb)
        drain(b)
        issue_cidx(i + 2, b)
        compute(i, b)

    pltpu.sync_copy(col_hbm.at[pl.ds(chunk_start(0), CH)], cidx_v.at[0])
    issue_rest(0, 0)
    issue_cidx(1, 1)

    def pair_body(p, _):
        step(2 * p, 0)
        step(2 * p + 1, 1)
        return 0

    lax.fori_loop(0, n_pair, pair_body, 0)

    wait_cidx(1)
    drain(0)

    pltpu.sync_copy(acc_v, out_hbm.at[pl.ds(row0, ROWS_W)])


def kernel(values, row_idx, col_idx, W, b):
    edges = jnp.arange(0, N + 1, ROWS_W, dtype=jnp.int32)
    bounds = jnp.searchsorted(row_idx, edges).astype(jnp.int32)
    bounds = jnp.concatenate(
        [bounds, jnp.full((48 - bounds.shape[0],), NNZ, jnp.int32)])
    return _spmm_sc(values, row_idx, col_idx, W, b, bounds)


# speedup vs baseline: 18.2906x; 1.0465x over previous
import functools

import jax
import jax.numpy as jnp
from jax import lax
from jax.experimental import pallas as pl
from jax.experimental.pallas import tpu as pltpu
from jax.experimental.pallas import tpu_sc as plsc

N = 16384
D_IN = 16384
D_OUT = 64
NNZ = 2621440

NW = 32
ROWS_W = N // NW
CH = 512
CHP = CH + 16
SUB = 128
NKV = D_OUT // 16
GRP = 8

_mesh = plsc.VectorSubcoreMesh(core_axis_name="c", subcore_axis_name="s")


@functools.partial(
    pl.kernel,
    out_type=jax.ShapeDtypeStruct((N, D_OUT), jnp.float32),
    mesh=_mesh,
    compiler_params=pltpu.CompilerParams(use_tc_tiling_on_sc=False),
    scratch_types=[
        pltpu.VMEM((48,), jnp.int32),
        pltpu.VMEM((2, CH), jnp.int32),
        pltpu.VMEM((2, CHP), jnp.int32),
        pltpu.VMEM((2, CHP), jnp.float32),
        pltpu.VMEM((2, CH, D_OUT), jnp.float32),
        pltpu.VMEM((ROWS_W, D_OUT), jnp.float32),
        pltpu.VMEM((D_OUT,), jnp.float32),
        pltpu.SemaphoreType.DMA,
        pltpu.SemaphoreType.DMA,
        pltpu.SemaphoreType.DMA,
        pltpu.SemaphoreType.DMA,
        pltpu.SemaphoreType.DMA,
        pltpu.SemaphoreType.DMA,
    ],
)
def _spmm_sc(values_hbm, row_hbm, col_hbm, w_hbm, b_hbm, bounds_hbm,
             out_hbm, bounds_v, cidx_v, ridx_v, vals_v, rows_v, acc_v,
             b_v, sem_g0, sem_g1, sem_s0, sem_s1, sem_c0, sem_c1):
    wid = lax.axis_index("s") * 2 + lax.axis_index("c")
    row0 = wid * ROWS_W

    pltpu.sync_copy(bounds_hbm, bounds_v)
    pltpu.sync_copy(b_hbm, b_v)

    bv = bounds_v[pl.ds(wid, 16)]
    lo = bv[0]
    hi = bv[1]
    lo8 = (lo // 8) * 8
    n_ch = (hi - lo8 + CH - 1) // CH
    n_pair = (n_ch + 1) // 2

    sem_g = (sem_g0, sem_g1)
    sem_s = (sem_s0, sem_s1)
    sem_c = (sem_c0, sem_c1)

    binit = [b_v[pl.ds(16 * k, 16)] for k in range(NKV)]

    @plsc.parallel_loop(0, ROWS_W)
    def _(r):
        for k in range(NKV):
            acc_v[r, pl.ds(16 * k, 16)] = binit[k]

    def chunk_start(i):
        return jnp.minimum(lo8 + i * CH, NNZ - CH)

    def issue_cidx(i, b):
        pltpu.async_copy(col_hbm.at[pl.ds(chunk_start(i), CH)],
                         cidx_v.at[b], sem_c[b])

    def wait_cidx(b):
        pltpu.make_async_copy(col_hbm.at[pl.ds(0, CH)], cidx_v.at[b],
                              sem_c[b]).wait()

    def issue_rest(i, b):
        s = chunk_start(i)
        for q in range(CH // SUB):
            pltpu.async_copy(
                w_hbm.at[cidx_v.at[b, pl.ds(q * SUB, SUB)]],
                rows_v.at[b, pl.ds(q * SUB, SUB)], sem_g[b])
        pltpu.async_copy(values_hbm.at[pl.ds(s, CH)],
                         vals_v.at[b, pl.ds(0, CH)], sem_s[b])
        pltpu.async_copy(row_hbm.at[pl.ds(s, CH)],
                         ridx_v.at[b, pl.ds(0, CH)], sem_s[b])

    def drain(b):
        for q in range(CH // SUB):
            pltpu.make_async_copy(
                w_hbm.at[pl.ds(0, SUB)],
                rows_v.at[b, pl.ds(q * SUB, SUB)], sem_g[b]).wait()
        pltpu.make_async_copy(values_hbm.at[pl.ds(0, CH)],
                              vals_v.at[b, pl.ds(0, CH)], sem_s[b]).wait()
        pltpu.make_async_copy(row_hbm.at[pl.ds(0, CH)],
                              ridx_v.at[b, pl.ds(0, CH)], sem_s[b]).wait()

    lane = lax.iota(jnp.int32, 16)

    def compute(i, b):
        s = chunk_start(i)
        start = lo8 + i * CH
        w_lo = jnp.maximum(lo, start) - s
        w_hi = jnp.minimum(hi, start + CH) - s

        @plsc.parallel_loop(0, CH, step=GRP)
        def _(j0):
            rclip = jnp.clip(ridx_v[b, pl.ds(j0, 16)] - row0,
                             0, ROWS_W - 1)
            gidx = j0 + lane
            vmask = (gidx >= w_lo) & (gidx < w_hi)
            vval = jnp.where(vmask, vals_v[b, pl.ds(j0, 16)], 0.0)
            for t in range(GRP):
                r = rclip[t]
                v = vval[t]
                for k in range(NKV):
                    plsc.addupdate(
                        acc_v.at[r, pl.ds(16 * k, 16)],
                        v * rows_v[b, j0 + t, pl.ds(16 * k, 16)])

    def step(i, b):
        nb = 1 - b
        wait_cidx(nb)
        issue_rest(i + 1, nb)
        drain(b)
        issue_cidx(i + 2, b)
        compute(i, b)

    pltpu.sync_copy(col_hbm.at[pl.ds(chunk_start(0), CH)], cidx_v.at[0])
    issue_rest(0, 0)
    issue_cidx(1, 1)

    def pair_body(p, _):
        step(2 * p, 0)
        step(2 * p + 1, 1)
        return 0

    lax.fori_loop(0, n_pair, pair_body, 0)

    wait_cidx(1)
    drain(0)

    pltpu.sync_copy(acc_v, out_hbm.at[pl.ds(row0, ROWS_W)])


def kernel(values, row_idx, col_idx, W, b):
    edges = jnp.arange(0, N + 1, ROWS_W, dtype=jnp.int32)
    bounds = jnp.searchsorted(row_idx, edges).astype(jnp.int32)
    bounds = jnp.concatenate(
        [bounds, jnp.full((48 - bounds.shape[0],), NNZ, jnp.int32)])
    return _spmm_sc(values, row_idx, col_idx, W, b, bounds)


# speedup vs baseline: 18.3819x; 1.0050x over previous
import functools

import jax
import jax.numpy as jnp
from jax import lax
from jax.experimental import pallas as pl
from jax.experimental.pallas import tpu as pltpu
from jax.experimental.pallas import tpu_sc as plsc

N = 16384
D_IN = 16384
D_OUT = 64
NNZ = 2621440

NW = 32
ROWS_W = N // NW
CH = 640
CHP = CH + 16
SUB = 128
NKV = D_OUT // 16
GRP = 8

_mesh = plsc.VectorSubcoreMesh(core_axis_name="c", subcore_axis_name="s")


@functools.partial(
    pl.kernel,
    out_type=jax.ShapeDtypeStruct((N, D_OUT), jnp.float32),
    mesh=_mesh,
    compiler_params=pltpu.CompilerParams(use_tc_tiling_on_sc=False),
    scratch_types=[
        pltpu.VMEM((48,), jnp.int32),
        pltpu.VMEM((2, CH), jnp.int32),
        pltpu.VMEM((2, CHP), jnp.int32),
        pltpu.VMEM((2, CHP), jnp.float32),
        pltpu.VMEM((2, CH, D_OUT), jnp.float32),
        pltpu.VMEM((ROWS_W, D_OUT), jnp.float32),
        pltpu.VMEM((D_OUT,), jnp.float32),
        pltpu.SemaphoreType.DMA,
        pltpu.SemaphoreType.DMA,
        pltpu.SemaphoreType.DMA,
        pltpu.SemaphoreType.DMA,
        pltpu.SemaphoreType.DMA,
        pltpu.SemaphoreType.DMA,
    ],
)
def _spmm_sc(values_hbm, row_hbm, col_hbm, w_hbm, b_hbm, bounds_hbm,
             out_hbm, bounds_v, cidx_v, ridx_v, vals_v, rows_v, acc_v,
             b_v, sem_g0, sem_g1, sem_s0, sem_s1, sem_c0, sem_c1):
    wid = lax.axis_index("s") * 2 + lax.axis_index("c")
    row0 = wid * ROWS_W

    pltpu.sync_copy(bounds_hbm, bounds_v)
    pltpu.sync_copy(b_hbm, b_v)

    bv = bounds_v[pl.ds(wid, 16)]
    lo = bv[0]
    hi = bv[1]
    lo8 = (lo // 8) * 8
    n_ch = (hi - lo8 + CH - 1) // CH
    n_pair = (n_ch + 1) // 2

    sem_g = (sem_g0, sem_g1)
    sem_s = (sem_s0, sem_s1)
    sem_c = (sem_c0, sem_c1)

    binit = [b_v[pl.ds(16 * k, 16)] for k in range(NKV)]

    @plsc.parallel_loop(0, ROWS_W)
    def _(r):
        for k in range(NKV):
            acc_v[r, pl.ds(16 * k, 16)] = binit[k]

    def chunk_start(i):
        return jnp.minimum(lo8 + i * CH, NNZ - CH)

    def issue_cidx(i, b):
        pltpu.async_copy(col_hbm.at[pl.ds(chunk_start(i), CH)],
                         cidx_v.at[b], sem_c[b])

    def wait_cidx(b):
        pltpu.make_async_copy(col_hbm.at[pl.ds(0, CH)], cidx_v.at[b],
                              sem_c[b]).wait()

    def issue_rest(i, b):
        s = chunk_start(i)
        for q in range(CH // SUB):
            pltpu.async_copy(
                w_hbm.at[cidx_v.at[b, pl.ds(q * SUB, SUB)]],
                rows_v.at[b, pl.ds(q * SUB, SUB)], sem_g[b])
        pltpu.async_copy(values_hbm.at[pl.ds(s, CH)],
                         vals_v.at[b, pl.ds(0, CH)], sem_s[b])
        pltpu.async_copy(row_hbm.at[pl.ds(s, CH)],
                         ridx_v.at[b, pl.ds(0, CH)], sem_s[b])

    def drain(b):
        for q in range(CH // SUB):
            pltpu.make_async_copy(
                w_hbm.at[pl.ds(0, SUB)],
                rows_v.at[b, pl.ds(q * SUB, SUB)], sem_g[b]).wait()
        pltpu.make_async_copy(values_hbm.at[pl.ds(0, CH)],
                              vals_v.at[b, pl.ds(0, CH)], sem_s[b]).wait()
        pltpu.make_async_copy(row_hbm.at[pl.ds(0, CH)],
                              ridx_v.at[b, pl.ds(0, CH)], sem_s[b]).wait()

    lane = lax.iota(jnp.int32, 16)

    def compute(i, b):
        s = chunk_start(i)
        start = lo8 + i * CH
        w_lo = jnp.maximum(lo, start) - s
        w_hi = jnp.minimum(hi, start + CH) - s

        @plsc.parallel_loop(0, CH, step=GRP)
        def _(j0):
            rclip = jnp.clip(ridx_v[b, pl.ds(j0, 16)] - row0,
                             0, ROWS_W - 1)
            gidx = j0 + lane
            vmask = (gidx >= w_lo) & (gidx < w_hi)
            vval = jnp.where(vmask, vals_v[b, pl.ds(j0, 16)], 0.0)
            for t in range(GRP):
                r = rclip[t]
                v = vval[t]
                for k in range(NKV):
                    plsc.addupdate(
                        acc_v.at[r, pl.ds(16 * k, 16)],
                        v * rows_v[b, j0 + t, pl.ds(16 * k, 16)])

    def step(i, b):
        nb = 1 - b
        wait_cidx(nb)
        issue_rest(i + 1, nb)
        drain(b)
        issue_cidx(i + 2, b)
        compute(i, b)

    pltpu.sync_copy(col_hbm.at[pl.ds(chunk_start(0), CH)], cidx_v.at[0])
    issue_rest(0, 0)
    issue_cidx(1, 1)

    def pair_body(p, _):
        step(2 * p, 0)
        step(2 * p + 1, 1)
        return 0

    lax.fori_loop(0, n_pair, pair_body, 0)

    wait_cidx(1)
    drain(0)

    pltpu.sync_copy(acc_v, out_hbm.at[pl.ds(row0, ROWS_W)])


def kernel(values, row_idx, col_idx, W, b):
    edges = jnp.arange(0, N + 1, ROWS_W, dtype=jnp.int32)
    bounds = jnp.searchsorted(row_idx, edges).astype(jnp.int32)
    bounds = jnp.concatenate(
        [bounds, jnp.full((48 - bounds.shape[0],), NNZ, jnp.int32)])
    return _spmm_sc(values, row_idx, col_idx, W, b, bounds)
